# baseline (speedup 1.0000x reference)
import functools

import jax
import jax.numpy as jnp
from jax import lax
from jax.experimental import pallas as pl
from jax.experimental.pallas import tpu as pltpu
from jax.experimental.pallas import tpu_sc as plsc

S = 2048
D = 1024
DA = D + 128
F = 1024
E = 8
K = 2
A = S * K
BM = 256
NB = A // BM + E
P = NB * BM
CHUNK = 512
NEG_INF = float("-inf")

NW = 32
TOK_W = S // NW
CH = 32
NCH = TOK_W // CH
IR = S // CH


def _route_body(x_ref, gw_ref, xw_ref, pos_ref, g_ref):
    x = x_ref[...]
    logits = lax.dot_general(
        x, gw_ref[...], (((1,), (1,)), ((), ())),
        preferred_element_type=jnp.float32)
    lane = lax.broadcasted_iota(jnp.int32, (S, E), 1).astype(jnp.float32)
    m1 = jnp.max(logits, axis=1, keepdims=True)
    i1 = jnp.min(jnp.where(logits == m1, lane, float(E)), axis=1,
                 keepdims=True)
    masked = jnp.where(lane == i1, NEG_INF, logits)
    m2 = jnp.max(masked, axis=1, keepdims=True)
    i2 = jnp.min(jnp.where(masked == m2, lane, float(E)), axis=1,
                 keepdims=True)
    z = jnp.exp(m2 - m1)
    denom = 1.0 + z
    w1 = 1.0 / denom
    w2 = z / denom

    h0 = (lane == i1).astype(jnp.float32)
    h1 = (lane == i2).astype(jnp.float32)

    lane_pad = lax.broadcasted_iota(jnp.int32, (S, DA - D), 1)
    is_w_col = (lane_pad == 0).astype(jnp.float32)
    xw_ref[0] = jnp.concatenate([w1 * x, w1 * is_w_col], axis=1)
    xw_ref[1] = jnp.concatenate([w2 * x, w2 * is_w_col], axis=1)

    r = lax.broadcasted_iota(jnp.int32, (CHUNK, CHUNK), 0)
    c = lax.broadcasted_iota(jnp.int32, (CHUNK, CHUNK), 1)
    tri = (c < r).astype(jnp.float32)

    run = jnp.zeros((1, E), jnp.float32)
    ranks = []
    for h in (h0, h1):
        for blk in range(S // CHUNK):
            hc = lax.slice(h, (blk * CHUNK, 0), ((blk + 1) * CHUNK, E))
            cum = lax.dot_general(
                tri, hc, (((1,), (0,)), ((), ())),
                preferred_element_type=jnp.float32)
            ranks.append(cum + run)
            run = run + jnp.sum(hc, axis=0, keepdims=True)
    rank0 = jnp.concatenate(ranks[: S // CHUNK], axis=0)
    rank1 = jnp.concatenate(ranks[S // CHUNK:], axis=0)

    counts = run.astype(jnp.int32)
    padded = ((counts + (BM - 1)) >> 8) << 8
    re_ = lax.broadcasted_iota(jnp.int32, (E, E), 0)
    ce_ = lax.broadcasted_iota(jnp.int32, (E, E), 1)
    tri_e = (re_ < ce_).astype(jnp.float32)
    start = lax.dot_general(
        padded.astype(jnp.float32), tri_e, (((1,), (0,)), ((), ())),
        preferred_element_type=jnp.float32)

    r0 = jnp.sum(h0 * rank0, axis=1, keepdims=True)
    r1 = jnp.sum(h1 * rank1, axis=1, keepdims=True)
    s0 = jnp.sum(h0 * start, axis=1, keepdims=True)
    s1 = jnp.sum(h1 * start, axis=1, keepdims=True)
    pos_ref[:, 0:1] = (s0 + r0).astype(jnp.int32)
    pos_ref[:, 1:2] = (s1 + r1).astype(jnp.int32)

    bs = start * (1.0 / BM)
    b_iota = lax.broadcasted_iota(jnp.int32, (1, NB), 1).astype(jnp.float32)
    acc = jnp.zeros((1, NB), jnp.int32)
    for e in range(E):
        bs_e = lax.slice(bs, (0, e), (1, e + 1))
        acc = acc + (b_iota >= bs_e).astype(jnp.int32)
    g_ref[...] = acc - 1


def _route_call(x, gate_w):
    return pl.pallas_call(
        _route_body,
        grid=(1,),
        in_specs=[
            pl.BlockSpec((S, D), lambda i: (0, 0)),
            pl.BlockSpec((E, D), lambda i: (0, 0)),
        ],
        out_specs=[
            pl.BlockSpec((K, S, DA), lambda i: (0, 0, 0)),
            pl.BlockSpec((S, K), lambda i: (0, 0)),
            pl.BlockSpec((1, NB), lambda i: (0, 0)),
        ],
        out_shape=[
            jax.ShapeDtypeStruct((K, S, DA), jnp.float32),
            jax.ShapeDtypeStruct((S, K), jnp.int32),
            jax.ShapeDtypeStruct((1, NB), jnp.int32),
        ],
        compiler_params=pltpu.CompilerParams(
            dimension_semantics=("arbitrary",)),
    )(x, gate_w)


@functools.lru_cache(maxsize=None)
def _make_dispatch():
    mesh = plsc.VectorSubcoreMesh(core_axis_name="c", subcore_axis_name="s")

    @functools.partial(
        pl.kernel,
        mesh=mesh,
        out_type=jax.ShapeDtypeStruct((P, DA), jnp.float32),
        scratch_types=[
            pltpu.VMEM((NCH, CH), jnp.int32),
            pltpu.VMEM((CH, DA), jnp.float32),
            pltpu.SemaphoreType.DMA,
        ],
    )
    def _dispatch(xw_hbm, pos_hbm, xs_hbm, idx_v, rows_v, sem):
        wid = lax.axis_index("s") * 2 + lax.axis_index("c")
        base = wid * TOK_W
        for k in range(K):
            pltpu.sync_copy(
                pos_hbm.at[pl.ds(k * IR + wid * NCH, NCH), :], idx_v)
            for c in range(NCH):
                pltpu.sync_copy(xw_hbm.at[k, pl.ds(base + c * CH, CH), :],
                                rows_v)
                pltpu.sync_copy(rows_v, xs_hbm.at[idx_v.at[c]])

    return _dispatch


def _gmm_body(g_sref, xs_ref, w_ref, b_ref, ys_ref):
    xs = xs_ref[...]
    y = lax.dot_general(
        xs[:, :D], w_ref[0], (((1,), (1,)), ((), ())),
        preferred_element_type=jnp.float32)
    ys_ref[...] = y + xs[:, D:D + 1] * b_ref[0]


def _gmm_call(g, xs, expert_w, expert_b):
    grid_spec = pltpu.PrefetchScalarGridSpec(
        num_scalar_prefetch=1,
        grid=(NB,),
        in_specs=[
            pl.BlockSpec((BM, DA), lambda b, g_ref: (b, 0)),
            pl.BlockSpec((1, F, D), lambda b, g_ref: (g_ref[b], 0, 0)),
            pl.BlockSpec((1, 1, F), lambda b, g_ref: (g_ref[b], 0, 0)),
        ],
        out_specs=pl.BlockSpec((BM, F), lambda b, g_ref: (b, 0)),
    )
    return pl.pallas_call(
        _gmm_body,
        grid_spec=grid_spec,
        out_shape=jax.ShapeDtypeStruct((P, F), jnp.float32),
        compiler_params=pltpu.CompilerParams(
            dimension_semantics=("arbitrary",)),
    )(g, xs, expert_w, expert_b.reshape(E, 1, F))


@functools.lru_cache(maxsize=None)
def _make_combine():
    mesh = plsc.VectorSubcoreMesh(core_axis_name="c", subcore_axis_name="s")

    @functools.partial(
        pl.kernel,
        mesh=mesh,
        out_type=jax.ShapeDtypeStruct((S, F), jnp.float32),
        scratch_types=[
            pltpu.VMEM((NCH, CH), jnp.int32),
            pltpu.VMEM((NCH, CH), jnp.int32),
            pltpu.VMEM((CH, F), jnp.float32),
            pltpu.VMEM((CH, F), jnp.float32),
            pltpu.SemaphoreType.DMA,
            pltpu.SemaphoreType.DMA,
        ],
    )
    def _combine(ys_hbm, pos_hbm, out_hbm, idx0_v, idx1_v, a_v, b_v, sem_a,
                 sem_b):
        wid = lax.axis_index("s") * 2 + lax.axis_index("c")
        base = wid * TOK_W
        pltpu.sync_copy(pos_hbm.at[pl.ds(wid * NCH, NCH), :], idx0_v)
        pltpu.sync_copy(pos_hbm.at[pl.ds(IR + wid * NCH, NCH), :], idx1_v)
        for c in range(NCH):
            cp_a = pltpu.async_copy(ys_hbm.at[idx0_v.at[c]], a_v, sem_a)
            cp_b = pltpu.async_copy(ys_hbm.at[idx1_v.at[c]], b_v, sem_b)
            cp_a.wait()
            cp_b.wait()
            for r in range(CH):
                def add_lanes(l, _, r=r):
                    sl = pl.ds(l * 16, 16)
                    a_v[r, sl] = a_v[r, sl] + b_v[r, sl]
                    return 0
                lax.fori_loop(0, F // 16, add_lanes, 0, unroll=4)
            pltpu.sync_copy(a_v, out_hbm.at[pl.ds(base + c * CH, CH), :])

    return _combine


def kernel(inputs, gate_w, expert_w, expert_b):
    B, S_, D_ = inputs.shape
    x = inputs.reshape(S, D)
    xw, pos, g = _route_call(x, gate_w)
    pos32 = pos.T.reshape(A // CH, CH)
    xs = _make_dispatch()(xw, pos32)
    ys = _gmm_call(g.reshape(NB), xs, expert_w, expert_b)
    out = _make_combine()(ys, pos32)
    return out.reshape(B, S, F)


# speedup vs baseline: 1.0187x; 1.0187x over previous
import functools

import jax
import jax.numpy as jnp
from jax import lax
from jax.experimental import pallas as pl
from jax.experimental.pallas import tpu as pltpu
from jax.experimental.pallas import tpu_sc as plsc

S = 2048
D = 1024
DA = D + 128
F = 1024
E = 8
K = 2
A = S * K
BM = 256
NB = A // BM + E
P = NB * BM
CHUNK = 512
NEG_INF = float("-inf")

NW = 32
TOK_W = S // NW
CH = 32
NCH = TOK_W // CH
IR = S // CH
CH_C = 16
NCH_C = TOK_W // CH_C
IR_C = S // CH_C


def _route_body(x_ref, gw_ref, xw_ref, pos_ref, g_ref):
    x = x_ref[...]
    logits = lax.dot_general(
        x, gw_ref[...], (((1,), (1,)), ((), ())),
        preferred_element_type=jnp.float32)
    lane = lax.broadcasted_iota(jnp.int32, (S, E), 1).astype(jnp.float32)
    m1 = jnp.max(logits, axis=1, keepdims=True)
    i1 = jnp.min(jnp.where(logits == m1, lane, float(E)), axis=1,
                 keepdims=True)
    masked = jnp.where(lane == i1, NEG_INF, logits)
    m2 = jnp.max(masked, axis=1, keepdims=True)
    i2 = jnp.min(jnp.where(masked == m2, lane, float(E)), axis=1,
                 keepdims=True)
    z = jnp.exp(m2 - m1)
    denom = 1.0 + z
    w1 = 1.0 / denom
    w2 = z / denom

    h0 = (lane == i1).astype(jnp.float32)
    h1 = (lane == i2).astype(jnp.float32)

    lane_pad = lax.broadcasted_iota(jnp.int32, (S, DA - D), 1)
    is_w_col = (lane_pad == 0).astype(jnp.float32)
    xw_ref[0] = jnp.concatenate([w1 * x, w1 * is_w_col], axis=1)
    xw_ref[1] = jnp.concatenate([w2 * x, w2 * is_w_col], axis=1)

    r = lax.broadcasted_iota(jnp.int32, (CHUNK, CHUNK), 0)
    c = lax.broadcasted_iota(jnp.int32, (CHUNK, CHUNK), 1)
    tri = (c < r).astype(jnp.float32)

    run = jnp.zeros((1, E), jnp.float32)
    ranks = []
    for h in (h0, h1):
        for blk in range(S // CHUNK):
            hc = lax.slice(h, (blk * CHUNK, 0), ((blk + 1) * CHUNK, E))
            cum = lax.dot_general(
                tri, hc, (((1,), (0,)), ((), ())),
                preferred_element_type=jnp.float32)
            ranks.append(cum + run)
            run = run + jnp.sum(hc, axis=0, keepdims=True)
    rank0 = jnp.concatenate(ranks[: S // CHUNK], axis=0)
    rank1 = jnp.concatenate(ranks[S // CHUNK:], axis=0)

    counts = run.astype(jnp.int32)
    padded = ((counts + (BM - 1)) >> 8) << 8
    re_ = lax.broadcasted_iota(jnp.int32, (E, E), 0)
    ce_ = lax.broadcasted_iota(jnp.int32, (E, E), 1)
    tri_e = (re_ < ce_).astype(jnp.float32)
    start = lax.dot_general(
        padded.astype(jnp.float32), tri_e, (((1,), (0,)), ((), ())),
        preferred_element_type=jnp.float32)

    r0 = jnp.sum(h0 * rank0, axis=1, keepdims=True)
    r1 = jnp.sum(h1 * rank1, axis=1, keepdims=True)
    s0 = jnp.sum(h0 * start, axis=1, keepdims=True)
    s1 = jnp.sum(h1 * start, axis=1, keepdims=True)
    pos_ref[:, 0:1] = (s0 + r0).astype(jnp.int32)
    pos_ref[:, 1:2] = (s1 + r1).astype(jnp.int32)

    bs = start * (1.0 / BM)
    b_iota = lax.broadcasted_iota(jnp.int32, (1, NB), 1).astype(jnp.float32)
    acc = jnp.zeros((1, NB), jnp.int32)
    for e in range(E):
        bs_e = lax.slice(bs, (0, e), (1, e + 1))
        acc = acc + (b_iota >= bs_e).astype(jnp.int32)
    g_ref[...] = acc - 1


def _route_call(x, gate_w):
    return pl.pallas_call(
        _route_body,
        grid=(1,),
        in_specs=[
            pl.BlockSpec((S, D), lambda i: (0, 0)),
            pl.BlockSpec((E, D), lambda i: (0, 0)),
        ],
        out_specs=[
            pl.BlockSpec((K, S, DA), lambda i: (0, 0, 0)),
            pl.BlockSpec((S, K), lambda i: (0, 0)),
            pl.BlockSpec((1, NB), lambda i: (0, 0)),
        ],
        out_shape=[
            jax.ShapeDtypeStruct((K, S, DA), jnp.float32),
            jax.ShapeDtypeStruct((S, K), jnp.int32),
            jax.ShapeDtypeStruct((1, NB), jnp.int32),
        ],
        compiler_params=pltpu.CompilerParams(
            dimension_semantics=("arbitrary",)),
    )(x, gate_w)


@functools.lru_cache(maxsize=None)
def _make_dispatch():
    mesh = plsc.VectorSubcoreMesh(core_axis_name="c", subcore_axis_name="s")

    @functools.partial(
        pl.kernel,
        mesh=mesh,
        out_type=jax.ShapeDtypeStruct((P, DA), jnp.float32),
        scratch_types=[
            pltpu.VMEM((K * NCH, CH), jnp.int32),
            pltpu.VMEM((CH, DA), jnp.float32),
            pltpu.VMEM((CH, DA), jnp.float32),
            pltpu.SemaphoreType.DMA,
            pltpu.SemaphoreType.DMA,
            pltpu.SemaphoreType.DMA,
            pltpu.SemaphoreType.DMA,
        ],
    )
    def _dispatch(xw_hbm, pos_hbm, xs_hbm, idx_v, row0_v, row1_v, sg0, sg1,
                  ss0, ss1):
        wid = lax.axis_index("s") * 2 + lax.axis_index("c")
        base = wid * TOK_W
        bufs = (row0_v, row1_v)
        gsems = (sg0, sg1)
        ssems = (ss0, ss1)
        pltpu.sync_copy(pos_hbm.at[pl.ds(wid * NCH, NCH), :],
                        idx_v.at[pl.ds(0, NCH)])
        pltpu.sync_copy(pos_hbm.at[pl.ds(IR + wid * NCH, NCH), :],
                        idx_v.at[pl.ds(NCH, NCH)])
        chunks = [(k, c) for k in range(K) for c in range(NCH)]
        gathers = {}
        scatters = {}
        for i in range(min(2, len(chunks))):
            k, c = chunks[i]
            gathers[i] = pltpu.async_copy(
                xw_hbm.at[k, pl.ds(base + c * CH, CH), :], bufs[i % 2],
                gsems[i % 2])
        for i, (k, c) in enumerate(chunks):
            if i >= 2:
                scatters[i - 2].wait()
                gathers[i] = pltpu.async_copy(
                    xw_hbm.at[k, pl.ds(base + c * CH, CH), :], bufs[i % 2],
                    gsems[i % 2])
            gathers[i].wait()
            scatters[i] = pltpu.async_copy(
                bufs[i % 2], xs_hbm.at[idx_v.at[k * NCH + c]], ssems[i % 2])
        scatters[len(chunks) - 2].wait()
        scatters[len(chunks) - 1].wait()

    return _dispatch


def _gmm_body(g_sref, xs_ref, w_ref, b_ref, ys_ref):
    xs = xs_ref[...]
    y = lax.dot_general(
        xs[:, :D], w_ref[0], (((1,), (1,)), ((), ())),
        preferred_element_type=jnp.float32)
    ys_ref[...] = y + xs[:, D:D + 1] * b_ref[0]


def _gmm_call(g, xs, expert_w, expert_b):
    grid_spec = pltpu.PrefetchScalarGridSpec(
        num_scalar_prefetch=1,
        grid=(NB,),
        in_specs=[
            pl.BlockSpec((BM, DA), lambda b, g_ref: (b, 0)),
            pl.BlockSpec((1, F, D), lambda b, g_ref: (g_ref[b], 0, 0)),
            pl.BlockSpec((1, 1, F), lambda b, g_ref: (g_ref[b], 0, 0)),
        ],
        out_specs=pl.BlockSpec((BM, F), lambda b, g_ref: (b, 0)),
    )
    return pl.pallas_call(
        _gmm_body,
        grid_spec=grid_spec,
        out_shape=jax.ShapeDtypeStruct((P, F), jnp.float32),
        compiler_params=pltpu.CompilerParams(
            dimension_semantics=("arbitrary",)),
    )(g, xs, expert_w, expert_b.reshape(E, 1, F))


@functools.lru_cache(maxsize=None)
def _make_combine():
    mesh = plsc.VectorSubcoreMesh(core_axis_name="c", subcore_axis_name="s")

    @functools.partial(
        pl.kernel,
        mesh=mesh,
        out_type=jax.ShapeDtypeStruct((S, F), jnp.float32),
        scratch_types=[
            pltpu.VMEM((NCH_C, CH_C), jnp.int32),
            pltpu.VMEM((NCH_C, CH_C), jnp.int32),
            pltpu.VMEM((CH_C, F), jnp.float32),
            pltpu.VMEM((CH_C, F), jnp.float32),
            pltpu.VMEM((CH_C, F), jnp.float32),
            pltpu.VMEM((CH_C, F), jnp.float32),
            pltpu.SemaphoreType.DMA,
            pltpu.SemaphoreType.DMA,
            pltpu.SemaphoreType.DMA,
            pltpu.SemaphoreType.DMA,
            pltpu.SemaphoreType.DMA,
            pltpu.SemaphoreType.DMA,
        ],
    )
    def _combine(ys_hbm, pos_hbm, out_hbm, idx0_v, idx1_v, a0_v, a1_v,
                 b0_v, b1_v, sa0, sa1, sb0, sb1, sw0, sw1):
        wid = lax.axis_index("s") * 2 + lax.axis_index("c")
        base = wid * TOK_W
        a_bufs = (a0_v, a1_v)
        b_bufs = (b0_v, b1_v)
        a_sems = (sa0, sa1)
        b_sems = (sb0, sb1)
        w_sems = (sw0, sw1)
        pltpu.sync_copy(pos_hbm.at[pl.ds(wid * NCH_C, NCH_C), :], idx0_v)
        pltpu.sync_copy(pos_hbm.at[pl.ds(IR_C + wid * NCH_C, NCH_C), :],
                        idx1_v)
        ga = {}
        gb = {}
        wr = {}
        for i in range(2):
            ga[i] = pltpu.async_copy(ys_hbm.at[idx0_v.at[i]], a_bufs[i],
                                     a_sems[i])
            gb[i] = pltpu.async_copy(ys_hbm.at[idx1_v.at[i]], b_bufs[i],
                                     b_sems[i])
        for i in range(NCH_C):
            s = i % 2
            if i >= 2:
                wr[i - 2].wait()
                ga[i] = pltpu.async_copy(ys_hbm.at[idx0_v.at[i]], a_bufs[s],
                                         a_sems[s])
                gb[i] = pltpu.async_copy(ys_hbm.at[idx1_v.at[i]], b_bufs[s],
                                         b_sems[s])
            ga[i].wait()
            gb[i].wait()
            a_v = a_bufs[s]
            b_v = b_bufs[s]
            for r in range(CH_C):
                def add_lanes(l, _, r=r, a_v=a_v, b_v=b_v):
                    sl = pl.ds(l * 16, 16)
                    a_v[r, sl] = a_v[r, sl] + b_v[r, sl]
                    return 0
                lax.fori_loop(0, F // 16, add_lanes, 0, unroll=2)
            wr[i] = pltpu.async_copy(
                a_v, out_hbm.at[pl.ds(base + i * CH_C, CH_C), :], w_sems[s])
        wr[NCH_C - 2].wait()
        wr[NCH_C - 1].wait()

    return _combine


def kernel(inputs, gate_w, expert_w, expert_b):
    B, S_, D_ = inputs.shape
    x = inputs.reshape(S, D)
    xw, pos, g = _route_call(x, gate_w)
    pos_flat = pos.T
    pos32 = pos_flat.reshape(A // CH, CH)
    pos16 = pos_flat.reshape(A // CH_C, CH_C)
    xs = _make_dispatch()(xw, pos32)
    ys = _gmm_call(g.reshape(NB), xs, expert_w, expert_b)
    out = _make_combine()(ys, pos16)
    return out.reshape(B, S, F)


# speedup vs baseline: 1.0281x; 1.0092x over previous
import functools

import jax
import jax.numpy as jnp
from jax import lax
from jax.experimental import pallas as pl
from jax.experimental.pallas import tpu as pltpu
from jax.experimental.pallas import tpu_sc as plsc

S = 2048
D = 1024
DA = D + 128
F = 1024
E = 8
K = 2
A = S * K
BM = 256
NB = A // BM + E
P = NB * BM
CHUNK = 512
NEG_INF = float("-inf")

NW = 32
TOK_W = S // NW
CH = 32
NCH = TOK_W // CH
IR = S // CH
CH_C = 16
NCH_C = TOK_W // CH_C
IR_C = S // CH_C


def _route_body(x_ref, gw_ref, xw_ref, pos_ref, g_ref):
    x = x_ref[...]
    logits = lax.dot_general(
        x, gw_ref[...], (((1,), (1,)), ((), ())),
        preferred_element_type=jnp.float32)
    lane = lax.broadcasted_iota(jnp.int32, (S, E), 1).astype(jnp.float32)
    m1 = jnp.max(logits, axis=1, keepdims=True)
    i1 = jnp.min(jnp.where(logits == m1, lane, float(E)), axis=1,
                 keepdims=True)
    masked = jnp.where(lane == i1, NEG_INF, logits)
    m2 = jnp.max(masked, axis=1, keepdims=True)
    i2 = jnp.min(jnp.where(masked == m2, lane, float(E)), axis=1,
                 keepdims=True)
    z = jnp.exp(m2 - m1)
    denom = 1.0 + z
    w1 = 1.0 / denom
    w2 = z / denom

    h0 = (lane == i1).astype(jnp.float32)
    h1 = (lane == i2).astype(jnp.float32)

    lane_pad = lax.broadcasted_iota(jnp.int32, (S, DA - D), 1)
    is_w_col = (lane_pad == 0).astype(jnp.float32)
    xw_ref[0] = jnp.concatenate([w1 * x, w1 * is_w_col], axis=1)
    xw_ref[1] = jnp.concatenate([w2 * x, w2 * is_w_col], axis=1)

    r = lax.broadcasted_iota(jnp.int32, (CHUNK, CHUNK), 0)
    c = lax.broadcasted_iota(jnp.int32, (CHUNK, CHUNK), 1)
    tri = (c < r).astype(jnp.float32)

    run = jnp.zeros((1, E), jnp.float32)
    ranks = []
    for h in (h0, h1):
        for blk in range(S // CHUNK):
            hc = lax.slice(h, (blk * CHUNK, 0), ((blk + 1) * CHUNK, E))
            cum = lax.dot_general(
                tri, hc, (((1,), (0,)), ((), ())),
                preferred_element_type=jnp.float32)
            ranks.append(cum + run)
            run = run + jnp.sum(hc, axis=0, keepdims=True)
    rank0 = jnp.concatenate(ranks[: S // CHUNK], axis=0)
    rank1 = jnp.concatenate(ranks[S // CHUNK:], axis=0)

    counts = run.astype(jnp.int32)
    padded = ((counts + (BM - 1)) >> 8) << 8
    re_ = lax.broadcasted_iota(jnp.int32, (E, E), 0)
    ce_ = lax.broadcasted_iota(jnp.int32, (E, E), 1)
    tri_e = (re_ < ce_).astype(jnp.float32)
    start = lax.dot_general(
        padded.astype(jnp.float32), tri_e, (((1,), (0,)), ((), ())),
        preferred_element_type=jnp.float32)

    r0 = jnp.sum(h0 * rank0, axis=1, keepdims=True)
    r1 = jnp.sum(h1 * rank1, axis=1, keepdims=True)
    s0 = jnp.sum(h0 * start, axis=1, keepdims=True)
    s1 = jnp.sum(h1 * start, axis=1, keepdims=True)
    pos_ref[:, 0:1] = (s0 + r0).astype(jnp.int32)
    pos_ref[:, 1:2] = (s1 + r1).astype(jnp.int32)

    bs = start * (1.0 / BM)
    b_iota = lax.broadcasted_iota(
        jnp.int32, (1, NB + 1), 1).astype(jnp.float32)
    acc = jnp.zeros((1, NB + 1), jnp.int32)
    for e in range(E):
        bs_e = lax.slice(bs, (0, e), (1, e + 1))
        acc = acc + (b_iota >= bs_e).astype(jnp.int32)
    nact = jnp.sum(padded, axis=1, keepdims=True) >> 8
    lane_nb = lax.broadcasted_iota(jnp.int32, (1, NB + 1), 1)
    g_ref[...] = jnp.where(lane_nb == NB, nact, acc - 1)


def _route_call(x, gate_w):
    return pl.pallas_call(
        _route_body,
        grid=(1,),
        in_specs=[
            pl.BlockSpec((S, D), lambda i: (0, 0)),
            pl.BlockSpec((E, D), lambda i: (0, 0)),
        ],
        out_specs=[
            pl.BlockSpec((K, S, DA), lambda i: (0, 0, 0)),
            pl.BlockSpec((S, K), lambda i: (0, 0)),
            pl.BlockSpec((1, NB + 1), lambda i: (0, 0)),
        ],
        out_shape=[
            jax.ShapeDtypeStruct((K, S, DA), jnp.float32),
            jax.ShapeDtypeStruct((S, K), jnp.int32),
            jax.ShapeDtypeStruct((1, NB + 1), jnp.int32),
        ],
        compiler_params=pltpu.CompilerParams(
            dimension_semantics=("arbitrary",)),
    )(x, gate_w)


@functools.lru_cache(maxsize=None)
def _make_dispatch():
    mesh = plsc.VectorSubcoreMesh(core_axis_name="c", subcore_axis_name="s")

    @functools.partial(
        pl.kernel,
        mesh=mesh,
        out_type=jax.ShapeDtypeStruct((P, DA), jnp.float32),
        scratch_types=[
            pltpu.VMEM((K * NCH_C, CH_C), jnp.int32),
            pltpu.VMEM((CH_C, DA), jnp.float32),
            pltpu.VMEM((CH_C, DA), jnp.float32),
            pltpu.VMEM((CH_C, DA), jnp.float32),
            pltpu.VMEM((CH_C, DA), jnp.float32),
            pltpu.SemaphoreType.DMA,
            pltpu.SemaphoreType.DMA,
            pltpu.SemaphoreType.DMA,
            pltpu.SemaphoreType.DMA,
            pltpu.SemaphoreType.DMA,
            pltpu.SemaphoreType.DMA,
            pltpu.SemaphoreType.DMA,
            pltpu.SemaphoreType.DMA,
        ],
    )
    def _dispatch(xw_hbm, pos_hbm, xs_hbm, idx_v, row0_v, row1_v, row2_v,
                  row3_v, sg0, sg1, sg2, sg3, ss0, ss1, ss2, ss3):
        wid = lax.axis_index("s") * 2 + lax.axis_index("c")
        base = wid * TOK_W
        bufs = (row0_v, row1_v, row2_v, row3_v)
        gsems = (sg0, sg1, sg2, sg3)
        ssems = (ss0, ss1, ss2, ss3)
        pltpu.sync_copy(pos_hbm.at[pl.ds(wid * NCH_C, NCH_C), :],
                        idx_v.at[pl.ds(0, NCH_C)])
        pltpu.sync_copy(pos_hbm.at[pl.ds(IR_C + wid * NCH_C, NCH_C), :],
                        idx_v.at[pl.ds(NCH_C, NCH_C)])
        chunks = [(k, c) for k in range(K) for c in range(NCH_C)]
        depth = 4
        gathers = {}
        scatters = {}
        for i in range(depth):
            k, c = chunks[i]
            gathers[i] = pltpu.async_copy(
                xw_hbm.at[k, pl.ds(base + c * CH_C, CH_C), :], bufs[i],
                gsems[i])
        for i, (k, c) in enumerate(chunks):
            s = i % depth
            if i >= depth:
                scatters[i - depth].wait()
                gathers[i] = pltpu.async_copy(
                    xw_hbm.at[k, pl.ds(base + c * CH_C, CH_C), :], bufs[s],
                    gsems[s])
            gathers[i].wait()
            scatters[i] = pltpu.async_copy(
                bufs[s], xs_hbm.at[idx_v.at[k * NCH_C + c]], ssems[s])
        for i in range(len(chunks) - depth, len(chunks)):
            scatters[i].wait()

    return _dispatch


def _gmm_body(g_sref, xs_ref, w_ref, b_ref, ys_ref):
    @pl.when(pl.program_id(0) < g_sref[NB])
    def _active():
        xs = xs_ref[...]
        y = lax.dot_general(
            xs[:, :D], w_ref[0], (((1,), (1,)), ((), ())),
            preferred_element_type=jnp.float32)
        ys_ref[...] = y + xs[:, D:D + 1] * b_ref[0]


def _gmm_call(g, xs, expert_w, expert_b):
    grid_spec = pltpu.PrefetchScalarGridSpec(
        num_scalar_prefetch=1,
        grid=(NB,),
        in_specs=[
            pl.BlockSpec((BM, DA), lambda b, g_ref: (b, 0)),
            pl.BlockSpec((1, F, D), lambda b, g_ref: (g_ref[b], 0, 0)),
            pl.BlockSpec((1, 1, F), lambda b, g_ref: (g_ref[b], 0, 0)),
        ],
        out_specs=pl.BlockSpec((BM, F), lambda b, g_ref: (b, 0)),
    )
    return pl.pallas_call(
        _gmm_body,
        grid_spec=grid_spec,
        out_shape=jax.ShapeDtypeStruct((P, F), jnp.float32),
        compiler_params=pltpu.CompilerParams(
            dimension_semantics=("arbitrary",)),
    )(g, xs, expert_w, expert_b.reshape(E, 1, F))


@functools.lru_cache(maxsize=None)
def _make_combine():
    mesh = plsc.VectorSubcoreMesh(core_axis_name="c", subcore_axis_name="s")

    @functools.partial(
        pl.kernel,
        mesh=mesh,
        out_type=jax.ShapeDtypeStruct((S, F), jnp.float32),
        scratch_types=[
            pltpu.VMEM((NCH_C, CH_C), jnp.int32),
            pltpu.VMEM((NCH_C, CH_C), jnp.int32),
            pltpu.VMEM((CH_C, F), jnp.float32),
            pltpu.VMEM((CH_C, F), jnp.float32),
            pltpu.VMEM((CH_C, F), jnp.float32),
            pltpu.VMEM((CH_C, F), jnp.float32),
            pltpu.SemaphoreType.DMA,
            pltpu.SemaphoreType.DMA,
            pltpu.SemaphoreType.DMA,
            pltpu.SemaphoreType.DMA,
            pltpu.SemaphoreType.DMA,
            pltpu.SemaphoreType.DMA,
        ],
    )
    def _combine(ys_hbm, pos_hbm, out_hbm, idx0_v, idx1_v, a0_v, a1_v,
                 b0_v, b1_v, sa0, sa1, sb0, sb1, sw0, sw1):
        wid = lax.axis_index("s") * 2 + lax.axis_index("c")
        base = wid * TOK_W
        a_bufs = (a0_v, a1_v)
        b_bufs = (b0_v, b1_v)
        a_sems = (sa0, sa1)
        b_sems = (sb0, sb1)
        w_sems = (sw0, sw1)
        pltpu.sync_copy(pos_hbm.at[pl.ds(wid * NCH_C, NCH_C), :], idx0_v)
        pltpu.sync_copy(pos_hbm.at[pl.ds(IR_C + wid * NCH_C, NCH_C), :],
                        idx1_v)
        ga = {}
        gb = {}
        wr = {}
        for i in range(2):
            ga[i] = pltpu.async_copy(ys_hbm.at[idx0_v.at[i]], a_bufs[i],
                                     a_sems[i])
            gb[i] = pltpu.async_copy(ys_hbm.at[idx1_v.at[i]], b_bufs[i],
                                     b_sems[i])
        for i in range(NCH_C):
            s = i % 2
            if i >= 2:
                wr[i - 2].wait()
                ga[i] = pltpu.async_copy(ys_hbm.at[idx0_v.at[i]], a_bufs[s],
                                         a_sems[s])
                gb[i] = pltpu.async_copy(ys_hbm.at[idx1_v.at[i]], b_bufs[s],
                                         b_sems[s])
            ga[i].wait()
            gb[i].wait()
            a_v = a_bufs[s]
            b_v = b_bufs[s]
            for r in range(CH_C):
                def add_lanes(l, _, r=r, a_v=a_v, b_v=b_v):
                    sl = pl.ds(l * 16, 16)
                    a_v[r, sl] = a_v[r, sl] + b_v[r, sl]
                    return 0
                lax.fori_loop(0, F // 16, add_lanes, 0, unroll=2)
            wr[i] = pltpu.async_copy(
                a_v, out_hbm.at[pl.ds(base + i * CH_C, CH_C), :], w_sems[s])
        wr[NCH_C - 2].wait()
        wr[NCH_C - 1].wait()

    return _combine


def kernel(inputs, gate_w, expert_w, expert_b):
    B, S_, D_ = inputs.shape
    x = inputs.reshape(S, D)
    xw, pos, g = _route_call(x, gate_w)
    pos16 = pos.T.reshape(A // CH_C, CH_C)
    xs = _make_dispatch()(xw, pos16)
    ys = _gmm_call(g.reshape(NB + 1), xs, expert_w, expert_b)
    out = _make_combine()(ys, pos16)
    return out.reshape(B, S, F)


# speedup vs baseline: 1.0302x; 1.0021x over previous
import functools

import jax
import jax.numpy as jnp
from jax import lax
from jax.experimental import pallas as pl
from jax.experimental.pallas import tpu as pltpu
from jax.experimental.pallas import tpu_sc as plsc

S = 2048
D = 1024
DA = D + 128
F = 1024
E = 8
K = 2
A = S * K
BM = 256
NB = A // BM + E
P = NB * BM
CHUNK = 512
NEG_INF = float("-inf")

NW = 32
TOK_W = S // NW
CH_C = 16
NCH_C = TOK_W // CH_C
IR_C = S // CH_C


def _route_body(x_ref, gw_ref, xw_ref, pos_ref, g_ref):
    x = x_ref[...]
    logits = lax.dot_general(
        x, gw_ref[...], (((1,), (1,)), ((), ())),
        preferred_element_type=jnp.float32)
    lane = lax.broadcasted_iota(jnp.int32, (S, E), 1).astype(jnp.float32)
    m1 = jnp.max(logits, axis=1, keepdims=True)
    i1 = jnp.min(jnp.where(logits == m1, lane, float(E)), axis=1,
                 keepdims=True)
    masked = jnp.where(lane == i1, NEG_INF, logits)
    m2 = jnp.max(masked, axis=1, keepdims=True)
    i2 = jnp.min(jnp.where(masked == m2, lane, float(E)), axis=1,
                 keepdims=True)
    z = jnp.exp(m2 - m1)
    denom = 1.0 + z
    w1 = 1.0 / denom
    w2 = z / denom

    h0 = (lane == i1).astype(jnp.float32)
    h1 = (lane == i2).astype(jnp.float32)

    lane_pad = lax.broadcasted_iota(jnp.int32, (S, DA - D), 1)
    is_w_col = (lane_pad == 0).astype(jnp.float32)
    xw_ref[0] = jnp.concatenate([w1 * x, w1 * is_w_col], axis=1)
    xw_ref[1] = jnp.concatenate([w2 * x, w2 * is_w_col], axis=1)

    r = lax.broadcasted_iota(jnp.int32, (CHUNK, CHUNK), 0)
    c = lax.broadcasted_iota(jnp.int32, (CHUNK, CHUNK), 1)
    tri = (c < r).astype(jnp.float32)

    run = jnp.zeros((1, E), jnp.float32)
    ranks = []
    for h in (h0, h1):
        for blk in range(S // CHUNK):
            hc = lax.slice(h, (blk * CHUNK, 0), ((blk + 1) * CHUNK, E))
            cum = lax.dot_general(
                tri, hc, (((1,), (0,)), ((), ())),
                preferred_element_type=jnp.float32)
            ranks.append(cum + run)
            run = run + jnp.sum(hc, axis=0, keepdims=True)
    rank0 = jnp.concatenate(ranks[: S // CHUNK], axis=0)
    rank1 = jnp.concatenate(ranks[S // CHUNK:], axis=0)

    counts = run.astype(jnp.int32)
    padded = ((counts + (BM - 1)) >> 8) << 8
    re_ = lax.broadcasted_iota(jnp.int32, (E, E), 0)
    ce_ = lax.broadcasted_iota(jnp.int32, (E, E), 1)
    tri_e = (re_ < ce_).astype(jnp.float32)
    start = lax.dot_general(
        padded.astype(jnp.float32), tri_e, (((1,), (0,)), ((), ())),
        preferred_element_type=jnp.float32)

    r0 = jnp.sum(h0 * rank0, axis=1, keepdims=True)
    r1 = jnp.sum(h1 * rank1, axis=1, keepdims=True)
    s0 = jnp.sum(h0 * start, axis=1, keepdims=True)
    s1 = jnp.sum(h1 * start, axis=1, keepdims=True)
    pos_ref[:, 0:1] = (s0 + r0).astype(jnp.int32)
    pos_ref[:, 1:2] = (s1 + r1).astype(jnp.int32)

    bs = start * (1.0 / BM)
    b_iota = lax.broadcasted_iota(
        jnp.int32, (1, NB + 1), 1).astype(jnp.float32)
    acc = jnp.zeros((1, NB + 1), jnp.int32)
    for e in range(E):
        bs_e = lax.slice(bs, (0, e), (1, e + 1))
        acc = acc + (b_iota >= bs_e).astype(jnp.int32)
    nact = jnp.sum(padded, axis=1, keepdims=True) >> 8
    lane_nb = lax.broadcasted_iota(jnp.int32, (1, NB + 1), 1)
    g_ref[...] = jnp.where(lane_nb == NB, nact, acc - 1)


def _route_call(x, gate_w):
    return pl.pallas_call(
        _route_body,
        grid=(1,),
        in_specs=[
            pl.BlockSpec((S, D), lambda i: (0, 0)),
            pl.BlockSpec((E, D), lambda i: (0, 0)),
        ],
        out_specs=[
            pl.BlockSpec((K, S, DA), lambda i: (0, 0, 0)),
            pl.BlockSpec((S, K), lambda i: (0, 0)),
            pl.BlockSpec((1, NB + 1), lambda i: (0, 0)),
        ],
        out_shape=[
            jax.ShapeDtypeStruct((K, S, DA), jnp.float32),
            jax.ShapeDtypeStruct((S, K), jnp.int32),
            jax.ShapeDtypeStruct((1, NB + 1), jnp.int32),
        ],
        compiler_params=pltpu.CompilerParams(
            dimension_semantics=("arbitrary",)),
    )(x, gate_w)


@functools.lru_cache(maxsize=None)
def _make_dispatch():
    mesh = plsc.VectorSubcoreMesh(core_axis_name="c", subcore_axis_name="s")

    @functools.partial(
        pl.kernel,
        mesh=mesh,
        out_type=jax.ShapeDtypeStruct((P, DA), jnp.float32),
        scratch_types=[
            pltpu.VMEM((K * NCH_C, CH_C), jnp.int32),
            pltpu.VMEM((CH_C, DA), jnp.float32),
            pltpu.VMEM((CH_C, DA), jnp.float32),
            pltpu.VMEM((CH_C, DA), jnp.float32),
            pltpu.VMEM((CH_C, DA), jnp.float32),
            pltpu.SemaphoreType.DMA,
            pltpu.SemaphoreType.DMA,
            pltpu.SemaphoreType.DMA,
            pltpu.SemaphoreType.DMA,
            pltpu.SemaphoreType.DMA,
            pltpu.SemaphoreType.DMA,
            pltpu.SemaphoreType.DMA,
            pltpu.SemaphoreType.DMA,
        ],
    )
    def _dispatch(xw_hbm, pos_hbm, xs_hbm, idx_v, row0_v, row1_v, row2_v,
                  row3_v, sg0, sg1, sg2, sg3, ss0, ss1, ss2, ss3):
        wid = lax.axis_index("s") * 2 + lax.axis_index("c")
        base = wid * TOK_W
        bufs = (row0_v, row1_v, row2_v, row3_v)
        gsems = (sg0, sg1, sg2, sg3)
        ssems = (ss0, ss1, ss2, ss3)
        pltpu.sync_copy(pos_hbm.at[pl.ds(wid * NCH_C, NCH_C), :],
                        idx_v.at[pl.ds(0, NCH_C)])
        pltpu.sync_copy(pos_hbm.at[pl.ds(IR_C + wid * NCH_C, NCH_C), :],
                        idx_v.at[pl.ds(NCH_C, NCH_C)])
        chunks = [(k, c) for k in range(K) for c in range(NCH_C)]
        depth = 4
        gathers = {}
        scatters = {}
        for i in range(depth):
            k, c = chunks[i]
            gathers[i] = pltpu.async_copy(
                xw_hbm.at[k, pl.ds(base + c * CH_C, CH_C), :], bufs[i],
                gsems[i])
        for i, (k, c) in enumerate(chunks):
            s = i % depth
            if i >= depth:
                scatters[i - depth].wait()
                gathers[i] = pltpu.async_copy(
                    xw_hbm.at[k, pl.ds(base + c * CH_C, CH_C), :], bufs[s],
                    gsems[s])
            gathers[i].wait()
            scatters[i] = pltpu.async_copy(
                bufs[s], xs_hbm.at[idx_v.at[k * NCH_C + c]], ssems[s])
        for i in range(len(chunks) - depth, len(chunks)):
            scatters[i].wait()

    return _dispatch


def _gmm_body(g_sref, xs_ref, w_ref, b_ref, ys_ref):
    @pl.when(pl.program_id(0) < g_sref[NB])
    def _active():
        xs = xs_ref[...]
        y = lax.dot_general(
            xs[:, :D], w_ref[0], (((1,), (1,)), ((), ())),
            preferred_element_type=jnp.float32)
        ys_ref[...] = y + xs[:, D:D + 1] * b_ref[0]


def _gmm_call(g, xs, expert_w, expert_b):
    grid_spec = pltpu.PrefetchScalarGridSpec(
        num_scalar_prefetch=1,
        grid=(NB,),
        in_specs=[
            pl.BlockSpec((BM, DA), lambda b, g_ref: (b, 0)),
            pl.BlockSpec((1, F, D), lambda b, g_ref: (g_ref[b], 0, 0)),
            pl.BlockSpec((1, 1, F), lambda b, g_ref: (g_ref[b], 0, 0)),
        ],
        out_specs=pl.BlockSpec((BM, F), lambda b, g_ref: (b, 0)),
    )
    return pl.pallas_call(
        _gmm_body,
        grid_spec=grid_spec,
        out_shape=jax.ShapeDtypeStruct((P, F), jnp.float32),
        compiler_params=pltpu.CompilerParams(
            dimension_semantics=("arbitrary",)),
    )(g, xs, expert_w, expert_b.reshape(E, 1, F))


@functools.lru_cache(maxsize=None)
def _make_combine():
    mesh = plsc.VectorSubcoreMesh(core_axis_name="c", subcore_axis_name="s")

    @functools.partial(
        pl.kernel,
        mesh=mesh,
        out_type=jax.ShapeDtypeStruct((S, F), jnp.float32),
        scratch_types=[
            pltpu.VMEM((NCH_C, CH_C), jnp.int32),
            pltpu.VMEM((NCH_C, CH_C), jnp.int32),
            pltpu.VMEM((CH_C, F), jnp.float32),
            pltpu.VMEM((CH_C, F), jnp.float32),
            pltpu.VMEM((CH_C, F), jnp.float32),
            pltpu.VMEM((CH_C, F), jnp.float32),
            pltpu.SemaphoreType.DMA,
            pltpu.SemaphoreType.DMA,
            pltpu.SemaphoreType.DMA,
            pltpu.SemaphoreType.DMA,
            pltpu.SemaphoreType.DMA,
            pltpu.SemaphoreType.DMA,
        ],
    )
    def _combine(ys_hbm, pos_hbm, out_hbm, idx0_v, idx1_v, a0_v, a1_v,
                 b0_v, b1_v, sa0, sa1, sb0, sb1, sw0, sw1):
        wid = lax.axis_index("s") * 2 + lax.axis_index("c")
        base = wid * TOK_W
        a_bufs = (a0_v, a1_v)
        b_bufs = (b0_v, b1_v)
        a_sems = (sa0, sa1)
        b_sems = (sb0, sb1)
        w_sems = (sw0, sw1)
        pltpu.sync_copy(pos_hbm.at[pl.ds(wid * NCH_C, NCH_C), :], idx0_v)
        pltpu.sync_copy(pos_hbm.at[pl.ds(IR_C + wid * NCH_C, NCH_C), :],
                        idx1_v)
        ga = {}
        gb = {}
        wr = {}
        for i in range(2):
            ga[i] = pltpu.async_copy(ys_hbm.at[idx0_v.at[i]], a_bufs[i],
                                     a_sems[i])
            gb[i] = pltpu.async_copy(ys_hbm.at[idx1_v.at[i]], b_bufs[i],
                                     b_sems[i])
        for i in range(NCH_C):
            s = i % 2
            if i >= 2:
                wr[i - 2].wait()
                ga[i] = pltpu.async_copy(ys_hbm.at[idx0_v.at[i]], a_bufs[s],
                                         a_sems[s])
                gb[i] = pltpu.async_copy(ys_hbm.at[idx1_v.at[i]], b_bufs[s],
                                         b_sems[s])
            ga[i].wait()
            gb[i].wait()
            a_v = a_bufs[s]
            b_v = b_bufs[s]
            for r in range(CH_C):
                def add_lanes(l, _, r=r, a_v=a_v, b_v=b_v):
                    sl = pl.ds(l * 16, 16)
                    a_v[r, sl] = a_v[r, sl] + b_v[r, sl]
                    return 0
                lax.fori_loop(0, F // 16, add_lanes, 0, unroll=2)
            wr[i] = pltpu.async_copy(
                a_v, out_hbm.at[pl.ds(base + i * CH_C, CH_C), :], w_sems[s])
        wr[NCH_C - 2].wait()
        wr[NCH_C - 1].wait()

    return _combine


def kernel(inputs, gate_w, expert_w, expert_b):
    B, S_, D_ = inputs.shape
    x = inputs.reshape(S, D)
    xw, pos, g = _route_call(x, gate_w)
    pos16 = pos.T.reshape(A // CH_C, CH_C)
    xs = _make_dispatch()(xw, pos16)
    ys = _gmm_call(g.reshape(NB + 1), xs, expert_w, expert_b)
    out = _make_combine()(ys, pos16)
    return out.reshape(B, S, F)
